# baseline (speedup 1.0000x reference)
import jax
import jax.numpy as jnp
from jax.experimental import pallas as pl
from jax.experimental.pallas import tpu as pltpu

_KNN = 5
_BOUND_LAMBDA = 1.0
_MAX_STEPS = 100
_CHUNK = 6
_NEG_BIG = -3.0e38


def _softmax(x):
    m = jnp.max(x, axis=1, keepdims=True)
    e = jnp.exp(x - m)
    return e / jnp.sum(e, axis=1, keepdims=True)


def _lame_kernel(scores_ref, feats_ref, out_ref, w_ref, unary_ref, y_ref):
    f = feats_ref[:]
    n = jnp.sqrt(jnp.sum(f * f, axis=1, keepdims=True))
    f = f / jnp.clip(n, 1e-12, None)

    G = jax.lax.dot_general(
        f, f, (((1,), (1,)), ((), ())), preferred_element_type=jnp.float32
    )
    N = G.shape[0]
    row_ids = jax.lax.broadcasted_iota(jnp.int32, (N, N), 0)
    col_ids = jax.lax.broadcasted_iota(jnp.int32, (N, N), 1)
    g = jnp.where(row_ids == col_ids, _NEG_BIG, G)

    W = jnp.zeros((N, N), jnp.float32)
    for _ in range(_KNN):
        m = jnp.max(g, axis=1, keepdims=True)
        cand = jnp.where(g == m, col_ids, N)
        idx = jnp.min(cand, axis=1, keepdims=True)
        hit = col_ids == idx
        W = W + hit.astype(jnp.float32)
        g = jnp.where(hit, _NEG_BIG, g)
    w_ref[:] = W

    unary = -jnp.log(scores_ref[:] + 1e-10)
    unary_ref[:] = unary
    y_ref[:] = _softmax(-unary)

    def cond_fn(state):
        i, _, done = state
        return jnp.logical_and(i < _MAX_STEPS, jnp.logical_not(done))

    def body_fn(state):
        i, oldE, done = state
        unary_v = unary_ref[:]
        W_v = w_ref[:]
        Y = y_ref[:]
        for _ in range(_CHUNK):
            pairwise = _BOUND_LAMBDA * jnp.dot(
                W_v, Y, preferred_element_type=jnp.float32
            )
            Ynew = _softmax(-unary_v + pairwise)
            E = jnp.sum(
                unary_v * Ynew
                - _BOUND_LAMBDA * pairwise * Ynew
                + Ynew * jnp.log(jnp.clip(Ynew, 1e-20, None))
            )
            active = jnp.logical_and(jnp.logical_not(done), i < _MAX_STEPS)
            newdone = jnp.logical_and(
                i > 1, jnp.abs(E - oldE) <= 1e-08 * jnp.abs(oldE)
            )
            Y = jnp.where(active, Ynew, Y)
            oldE = jnp.where(active, E, oldE)
            done = jnp.where(active, newdone, done)
            i = jnp.where(active, i + 1, i)
        y_ref[:] = Y
        return (i, oldE, done)

    state0 = (jnp.int32(0), jnp.array(jnp.inf, dtype=jnp.float32), jnp.array(False))
    jax.lax.while_loop(cond_fn, body_fn, state0)
    out_ref[:] = y_ref[:]


def kernel(scores_raw, feats):
    B, C, H, Wd = scores_raw.shape
    scores = scores_raw.reshape(-1, H * Wd)
    f = feats.reshape(feats.shape[:-3] + (-1,))
    if f.shape[0] == 1:
        f = jnp.squeeze(f, 0)
    M, L = scores.shape
    return pl.pallas_call(
        _lame_kernel,
        out_shape=jax.ShapeDtypeStruct((M, L), jnp.float32),
        scratch_shapes=[
            pltpu.VMEM((M, M), jnp.float32),
            pltpu.VMEM((M, L), jnp.float32),
            pltpu.VMEM((M, L), jnp.float32),
        ],
    )(scores, f)


# speedup vs baseline: 1.1746x; 1.1746x over previous
import jax
import jax.numpy as jnp
from jax.experimental import pallas as pl
from jax.experimental.pallas import tpu as pltpu

_KNN = 5
_BOUND_LAMBDA = 1.0
_MAX_STEPS = 100
_UNROLL = 10
_NEG_BIG = -3.0e38


def _softmax(x):
    m = jnp.max(x, axis=1, keepdims=True)
    e = jnp.exp(x - m)
    return e / jnp.sum(e, axis=1, keepdims=True)


def _step(W, unary, Y, i, oldE, done):
    pairwise = _BOUND_LAMBDA * jnp.dot(W, Y, preferred_element_type=jnp.float32)
    Ynew = _softmax(-unary + pairwise)
    E = jnp.sum(
        unary * Ynew
        - _BOUND_LAMBDA * pairwise * Ynew
        + Ynew * jnp.log(jnp.clip(Ynew, 1e-20, None))
    )
    active = jnp.logical_and(jnp.logical_not(done), i < _MAX_STEPS)
    newdone = jnp.logical_and(i > 1, jnp.abs(E - oldE) <= 1e-08 * jnp.abs(oldE))
    Y = jnp.where(active, Ynew, Y)
    oldE = jnp.where(active, E, oldE)
    done = jnp.where(active, newdone, done)
    i = jnp.where(active, i + 1, i)
    return Y, i, oldE, done


def _lame_kernel(scores_ref, feats_ref, out_ref, w_ref, unary_ref, y_ref):
    f = feats_ref[:]
    n = jnp.sqrt(jnp.sum(f * f, axis=1, keepdims=True))
    f = f / jnp.clip(n, 1e-12, None)

    G = jax.lax.dot_general(
        f, f, (((1,), (1,)), ((), ())), preferred_element_type=jnp.float32
    )
    N = G.shape[0]
    row_ids = jax.lax.broadcasted_iota(jnp.int32, (N, N), 0)
    col_ids = jax.lax.broadcasted_iota(jnp.int32, (N, N), 1)
    g = jnp.where(row_ids == col_ids, _NEG_BIG, G)

    W = jnp.zeros((N, N), jnp.float32)
    for _ in range(_KNN):
        m = jnp.max(g, axis=1, keepdims=True)
        cand = jnp.where(g == m, col_ids, N)
        idx = jnp.min(cand, axis=1, keepdims=True)
        hit = col_ids == idx
        W = W + hit.astype(jnp.float32)
        g = jnp.where(hit, _NEG_BIG, g)
    w_ref[:] = W

    unary = -jnp.log(scores_ref[:] + 1e-10)
    unary_ref[:] = unary
    Y = _softmax(-unary)

    i = jnp.int32(0)
    oldE = jnp.array(jnp.inf, dtype=jnp.float32)
    done = jnp.array(False)
    for _ in range(_UNROLL):
        Y, i, oldE, done = _step(W, unary, Y, i, oldE, done)
    y_ref[:] = Y

    def cond_fn(state):
        i, _, done = state
        return jnp.logical_and(i < _MAX_STEPS, jnp.logical_not(done))

    def body_fn(state):
        i, oldE, done = state
        Yc = y_ref[:]
        Yc, i, oldE, done = _step(w_ref[:], unary_ref[:], Yc, i, oldE, done)
        y_ref[:] = Yc
        return (i, oldE, done)

    jax.lax.while_loop(cond_fn, body_fn, (i, oldE, done))
    out_ref[:] = y_ref[:]


def kernel(scores_raw, feats):
    B, C, H, Wd = scores_raw.shape
    scores = scores_raw.reshape(-1, H * Wd)
    f = feats.reshape(feats.shape[:-3] + (-1,))
    if f.shape[0] == 1:
        f = jnp.squeeze(f, 0)
    M, L = scores.shape
    return pl.pallas_call(
        _lame_kernel,
        out_shape=jax.ShapeDtypeStruct((M, L), jnp.float32),
        scratch_shapes=[
            pltpu.VMEM((M, M), jnp.float32),
            pltpu.VMEM((M, L), jnp.float32),
            pltpu.VMEM((M, L), jnp.float32),
        ],
    )(scores, f)
